# Initial kernel scaffold; baseline (speedup 1.0000x reference)
import jax
import jax.numpy as jnp
from jax.experimental import pallas as pl


def kernel(input_embs, input_seq_lengths, beg_seq_param):
    raise NotImplementedError("write your pallas kernel here")


# speedup vs baseline: 2.5007x; 2.5007x over previous
import math

import jax
import jax.numpy as jnp
import numpy as np
from jax import lax
from jax.experimental import pallas as pl
from jax.experimental.pallas import tpu as pltpu
from jax.experimental.pallas import tpu_sc as plsc

HIDDEN = 512
PAD_MULT = 128
EXTRA = 1
MAX_LEN = 2176
LANES = 16
CH = 64


def _sinusoidal_pos_emb(max_len: int, d: int) -> np.ndarray:
    pos = np.arange(max_len, dtype=np.float32)[:, None]
    i = np.arange(0, d, 2, dtype=np.float32)
    div = np.exp(-math.log(10000.0) * i / d)
    pe = np.zeros((max_len, d), dtype=np.float32)
    pe[:, 0::2] = np.sin(pos * div)
    pe[:, 1::2] = np.cos(pos * div)
    return pe


def kernel(input_embs, input_seq_lengths, beg_seq_param):
    T, D = input_embs.shape
    B = input_seq_lengths.shape[0]
    ML = MAX_LEN
    scale = jnp.float32(math.sqrt(D))

    mesh = plsc.VectorSubcoreMesh(core_axis_name="c", subcore_axis_name="s")
    NC, NS = mesh.num_cores, mesh.num_subcores
    NW = NC * NS
    n_chunks = ML // CH
    n_extra = n_chunks - NW
    assert ML % CH == 0 and 0 <= n_extra <= NW

    pe_tab = jnp.asarray(_sinusoidal_pos_emb(ML, D))

    def body(x_hbm, len_hbm, st_hbm, beg_hbm, pe_hbm, out_hbm,
             len_v, st_v, pe_v, x_v, idx_v, beg_v, sem):
        cid = lax.axis_index("c")
        sid = lax.axis_index("s")
        w = sid * NC + cid

        pltpu.sync_copy(len_hbm, len_v.at[pl.ds(0, B)])
        pltpu.sync_copy(st_hbm, st_v.at[pl.ds(0, B)])

        iota = lax.iota(jnp.int32, LANES)

        pltpu.sync_copy(beg_hbm, beg_v)
        for k in range(D // LANES):
            sl = pl.ds(k * LANES, LANES)
            beg_v[sl] = beg_v[sl] * scale

        def run_chunk(c):
            p0 = (c * CH).astype(jnp.int32)
            pltpu.sync_copy(pe_hbm.at[pl.ds(p0, CH)], pe_v)

            def seq_body(b, carry):
                len_b = len_v[pl.ds(b, LANES)][0]
                st_b = st_v[pl.ds(b, LANES)][0]

                @pl.when(p0 > len_b)
                def _():
                    pltpu.sync_copy(pe_v, out_hbm.at[b, pl.ds(p0, CH)])

                @pl.when(p0 <= len_b)
                def _():
                    base = st_b + p0 - 1
                    for j in range(CH // LANES):
                        v = jnp.clip(base + j * LANES + iota, 0, T - 1)
                        idx_v[pl.ds(j * LANES, LANES)] = v

                    pltpu.async_copy(x_hbm.at[idx_v], x_v, sem).wait()

                    def row_body(r, rc):
                        p = p0 + r
                        valid = jnp.logical_and(p >= 1, p <= len_b)
                        m = jnp.where(valid, scale, jnp.float32(0.0))
                        for k in range(D // LANES):
                            sl = pl.ds(k * LANES, LANES)
                            x_v[r, sl] = x_v[r, sl] * m + pe_v[r, sl]
                        return rc

                    lax.fori_loop(0, CH, row_body, jnp.int32(0))

                    @pl.when(p0 == 0)
                    def _():
                        for k in range(D // LANES):
                            sl = pl.ds(k * LANES, LANES)
                            x_v[0, sl] = x_v[0, sl] + beg_v[sl]

                    pltpu.sync_copy(x_v, out_hbm.at[b, pl.ds(p0, CH)])

                return carry

            lax.fori_loop(0, B, seq_body, jnp.int32(0))

        run_chunk(w)

        @pl.when(w < n_extra)
        def _():
            run_chunk(w + NW)

    fn = pl.kernel(
        body,
        out_type=jax.ShapeDtypeStruct((B, ML, D), jnp.float32),
        mesh=mesh,
        scratch_types=[
            pltpu.VMEM((B + LANES,), jnp.int32),
            pltpu.VMEM((B + LANES,), jnp.int32),
            pltpu.VMEM((CH, D), jnp.float32),
            pltpu.VMEM((CH, D), jnp.float32),
            pltpu.VMEM((CH,), jnp.int32),
            pltpu.VMEM((D,), jnp.float32),
            pltpu.SemaphoreType.DMA,
        ],
    )
    lengths = input_seq_lengths.astype(jnp.int32)
    starts = jnp.concatenate([jnp.zeros((1,), jnp.int32),
                              jnp.cumsum(lengths)[:-1].astype(jnp.int32)])
    return fn(input_embs, lengths, starts, beg_seq_param, pe_tab)


# speedup vs baseline: 4.3810x; 1.7519x over previous
import math

import jax
import jax.numpy as jnp
import numpy as np
from jax import lax
from jax.experimental import pallas as pl
from jax.experimental.pallas import tpu as pltpu
from jax.experimental.pallas import tpu_sc as plsc

HIDDEN = 512
PAD_MULT = 128
EXTRA = 1
MAX_LEN = 2176
LANES = 16
CH = 32

_SCHED_LENGTHS = (2048, 512, 1024, 1536, 768, 1280, 896, 1152,
                  640, 1408, 1024, 704, 960, 832, 1088, 512)


def _sinusoidal_pos_emb(max_len: int, d: int) -> np.ndarray:
    pos = np.arange(max_len, dtype=np.float32)[:, None]
    i = np.arange(0, d, 2, dtype=np.float32)
    div = np.exp(-math.log(10000.0) * i / d)
    pe = np.zeros((max_len, d), dtype=np.float32)
    pe[:, 0::2] = np.sin(pos * div)
    pe[:, 1::2] = np.cos(pos * div)
    return pe


def _chunk_schedule(n_chunks: int, n_workers: int) -> np.ndarray:
    weights = []
    for c in range(n_chunks):
        g = sum(1 for L in _SCHED_LENGTHS if c * CH <= L)
        weights.append(16 + g)
    order = sorted(range(n_chunks), key=lambda c: -weights[c])
    loads = [0] * n_workers
    bins = [[] for _ in range(n_workers)]
    for c in order:
        w = min(range(n_workers), key=lambda i: (loads[i], len(bins[i])))
        bins[w].append(c)
        loads[w] += weights[c]
    k = max(len(b) for b in bins)
    sched = np.full((n_workers, 1, k + LANES), -1, dtype=np.int32)
    for w, b in enumerate(bins):
        sched[w, 0, :len(b)] = b
    return sched


def kernel(input_embs, input_seq_lengths, beg_seq_param):
    T, D = input_embs.shape
    B = input_seq_lengths.shape[0]
    ML = MAX_LEN
    scale = jnp.float32(math.sqrt(D))

    mesh = plsc.VectorSubcoreMesh(core_axis_name="c", subcore_axis_name="s")
    NC, NS = mesh.num_cores, mesh.num_subcores
    NW = NC * NS
    n_chunks = ML // CH
    assert ML % CH == 0
    NV = D // LANES

    pe_tab = jnp.asarray(_sinusoidal_pos_emb(ML, D))
    sched_tab = jnp.asarray(_chunk_schedule(n_chunks, NW))
    SW = sched_tab.shape[2]
    K = SW - LANES

    lengths = input_seq_lengths.astype(jnp.int32)
    starts = jnp.concatenate([jnp.zeros((1,), jnp.int32),
                              jnp.cumsum(lengths)[:-1].astype(jnp.int32)])
    cond = (CH * jnp.arange(n_chunks, dtype=jnp.int32)[:, None]) <= lengths[None, :]
    gcnt = jnp.sum(cond, axis=1).astype(jnp.int32)
    ordr = jnp.argsort(jnp.logical_not(cond), axis=1, stable=True).astype(jnp.int32)
    blist = jnp.concatenate(
        [gcnt[:, None], ordr,
         jnp.full((n_chunks, LANES - 1), -1, jnp.int32)], axis=1)[:, None, :]

    def body(x_hbm, len_hbm, st_hbm, beg_hbm, pe_hbm, sched_hbm, bl_hbm,
             out_hbm,
             len_v, st_v, pe_v, xa_v, xb_v, idxa_v, idxb_v, blv, schv, beg_v,
             sg_a, sg_b, ss_a, ss_b, s_pe):
        cid = lax.axis_index("c")
        sid = lax.axis_index("s")
        w = sid * NC + cid

        pltpu.sync_copy(len_hbm, len_v.at[pl.ds(0, B)])
        pltpu.sync_copy(st_hbm, st_v.at[pl.ds(0, B)])
        pltpu.sync_copy(sched_hbm.at[w, 0], schv)

        iota = lax.iota(jnp.int32, LANES)

        pltpu.sync_copy(beg_hbm, beg_v)
        for k in range(NV):
            sl = pl.ds(k * LANES, LANES)
            beg_v[sl] = beg_v[sl] * scale

        bufs = ((xa_v, idxa_v, sg_a, ss_a), (xb_v, idxb_v, sg_b, ss_b))

        def get_b(i):
            return blv[pl.ds(i + 1, LANES)][0]

        def issue_gather(i, par):
            x_v, idx_v, sg, _ = bufs[par]
            b = get_b(i)
            st_b = st_v[pl.ds(b, LANES)][0]
            base = st_b + get_p0() - 1
            for j in range(CH // LANES):
                idx_v[pl.ds(j * LANES, LANES)] = jnp.clip(
                    base + j * LANES + iota, 0, T - 1)
            pltpu.async_copy(x_hbm.at[idx_v], x_v, sg)

        p0_box = []

        def get_p0():
            return p0_box[0]

        def compute_store(i, par):
            x_v, idx_v, sg, ss = bufs[par]
            b = get_b(i)
            len_b = len_v[pl.ds(b, LANES)][0]
            p0 = get_p0()
            pltpu.make_async_copy(x_hbm.at[idx_v], x_v, sg).wait()

            def row_body(r, rc):
                p = p0 + r
                valid = jnp.logical_and(p >= 1, p <= len_b)
                m = jnp.where(valid, scale, jnp.float32(0.0))
                for k in range(NV):
                    sl = pl.ds(k * LANES, LANES)
                    x_v[r, sl] = x_v[r, sl] * m + pe_v[r, sl]
                return rc

            lax.fori_loop(0, CH, row_body, jnp.int32(0))

            @pl.when(p0 == 0)
            def _():
                for k in range(NV):
                    sl = pl.ds(k * LANES, LANES)
                    x_v[0, sl] = x_v[0, sl] + beg_v[sl]

            pltpu.async_copy(x_v, out_hbm.at[b, pl.ds(p0, CH)], ss)

        def run_slot(j, carry):
            c = schv[pl.ds(j, LANES)][0]

            @pl.when(c >= 0)
            def _():
                p0 = c * CH
                p0_box.clear()
                p0_box.append(p0)
                pltpu.sync_copy(pe_hbm.at[pl.ds(p0, CH)], pe_v)
                pltpu.sync_copy(bl_hbm.at[c, 0], blv)
                gcnt_c = blv[pl.ds(0, LANES)][0]

                def pe_body(i, pc):
                    b = get_b(i)
                    pltpu.async_copy(pe_v, out_hbm.at[b, pl.ds(p0, CH)], s_pe)
                    return pc

                lax.fori_loop(gcnt_c, B, pe_body, jnp.int32(0))

                @pl.when(gcnt_c > 0)
                def _():
                    issue_gather(0, 0)

                def pipe_body(i, pc):
                    @pl.when(i % 2 == 0)
                    def _():
                        _stage(i, 0)

                    @pl.when(i % 2 == 1)
                    def _():
                        _stage(i, 1)

                    return pc

                def _stage(i, par):
                    nxt = 1 - par

                    @pl.when(i + 1 < gcnt_c)
                    def _():
                        @pl.when(i >= 1)
                        def _():
                            _, _, _, ss_n = bufs[nxt]
                            pltpu.make_async_copy(
                                bufs[nxt][0], out_hbm.at[0, pl.ds(0, CH)],
                                ss_n).wait()

                        issue_gather(i + 1, nxt)

                    compute_store(i, par)

                lax.fori_loop(0, gcnt_c, pipe_body, jnp.int32(0))

                def drain_store(par):
                    pltpu.make_async_copy(
                        bufs[par][0], out_hbm.at[0, pl.ds(0, CH)],
                        bufs[par][3]).wait()

                for want in (2, 1):
                    @pl.when(jnp.logical_and(gcnt_c >= want,
                                             (gcnt_c - want) % 2 == 0))
                    def _():
                        drain_store(0)

                    @pl.when(jnp.logical_and(gcnt_c >= want,
                                             (gcnt_c - want) % 2 == 1))
                    def _():
                        drain_store(1)

                def pe_drain(i, pc):
                    pltpu.make_async_copy(
                        pe_v, out_hbm.at[0, pl.ds(0, CH)], s_pe).wait()
                    return pc

                lax.fori_loop(gcnt_c, B, pe_drain, jnp.int32(0))

            return carry

        lax.fori_loop(0, K, run_slot, jnp.int32(0))

    fn = pl.kernel(
        body,
        out_type=jax.ShapeDtypeStruct((B, ML, D), jnp.float32),
        mesh=mesh,
        scratch_types=[
            pltpu.VMEM((B + LANES,), jnp.int32),
            pltpu.VMEM((B + LANES,), jnp.int32),
            pltpu.VMEM((CH, D), jnp.float32),
            pltpu.VMEM((CH, D), jnp.float32),
            pltpu.VMEM((CH, D), jnp.float32),
            pltpu.VMEM((CH,), jnp.int32),
            pltpu.VMEM((CH,), jnp.int32),
            pltpu.VMEM((2 * LANES,), jnp.int32),
            pltpu.VMEM((SW,), jnp.int32),
            pltpu.VMEM((D,), jnp.float32),
            pltpu.SemaphoreType.DMA,
            pltpu.SemaphoreType.DMA,
            pltpu.SemaphoreType.DMA,
            pltpu.SemaphoreType.DMA,
            pltpu.SemaphoreType.DMA,
        ],
    )
    return fn(input_embs, lengths, starts, beg_seq_param, pe_tab,
              sched_tab, blist)
